# Initial kernel scaffold; baseline (speedup 1.0000x reference)
import jax
import jax.numpy as jnp
from jax.experimental import pallas as pl


def kernel(features, multi_r_data, batch_nodes, W1, b1, g1, beta1, W2, b2, g2, beta2):
    raise NotImplementedError("write your pallas kernel here")


# speedup vs baseline: 10.4070x; 10.4070x over previous
import functools
import math

import jax
import jax.numpy as jnp
from jax import lax
from jax.experimental import pallas as pl
from jax.experimental.pallas import tpu as pltpu
from jax.experimental.pallas import tpu_sc as plsc

N = 10000
E = 320000
R = 3
F_IN = 128
HID = 128
OUT = 64
B = 1024
EPS = 1e-5
BN_SCALE = 1.0 / math.sqrt(1.0 + EPS)

NC = 2
NS = 16
CH = 80
EPC = E // NC
EPW = EPC // NS
NCH = EPW // CH
BPW = B // (NC * NS)

_mesh = plsc.VectorSubcoreMesh(core_axis_name="c", subcore_axis_name="s")


@functools.partial(
    pl.kernel, mesh=_mesh,
    out_type=jax.ShapeDtypeStruct((NC, R, N, HID), jnp.float32),
    scratch_types=[
        pltpu.VMEM_SHARED((N, HID), jnp.float32),
        pltpu.VMEM((CH, HID), jnp.float32),
        pltpu.VMEM((CH,), jnp.int32),
    ],
)
def _deg_kernel(edges_hbm, ones_hbm, zeros_hbm, out_hbm, deg_sp, ones_v, idx_v):
    c = lax.axis_index("c")
    s = lax.axis_index("s")
    pltpu.sync_copy(ones_hbm, ones_v)
    base = c * EPC + s * EPW
    for r in range(R):
        @pl.when(s == 0)
        def _():
            pltpu.sync_copy(zeros_hbm, deg_sp)
        plsc.subcore_barrier()

        def body(i, carry):
            off = pl.multiple_of((r * 2 + 1) * E + base + i * CH, 8)
            pltpu.sync_copy(edges_hbm.at[pl.ds(off, CH)], idx_v)
            pltpu.sync_copy(ones_v, deg_sp.at[idx_v], add=True)
            return carry

        lax.fori_loop(0, NCH, body, 0)
        plsc.subcore_barrier()

        @pl.when(s == 0)
        def _():
            pltpu.sync_copy(deg_sp, out_hbm.at[c, r])
        plsc.subcore_barrier()


def _make_agg(D):
    @functools.partial(
        pl.kernel, mesh=_mesh,
        out_type=jax.ShapeDtypeStruct((NC, R, N, D), jnp.float32),
        scratch_types=[
            pltpu.VMEM_SHARED((N, D), jnp.float32),
            pltpu.VMEM((CH, D), jnp.float32),
            pltpu.VMEM((CH,), jnp.int32),
            pltpu.VMEM((CH,), jnp.int32),
            pltpu.SemaphoreType.DMA,
        ],
    )
    def agg(hs_hbm, edges_hbm, zeros_hbm, out_hbm, acc_sp, rows_v, sidx_v,
            didx_v, sem):
        c = lax.axis_index("c")
        s = lax.axis_index("s")
        base = c * EPC + s * EPW
        for r in range(R):
            @pl.when(s == 0)
            def _():
                pltpu.sync_copy(zeros_hbm, acc_sp)
            plsc.subcore_barrier()

            def body(i, carry):
                off_s = pl.multiple_of(r * 2 * E + base + i * CH, 8)
                off_d = pl.multiple_of((r * 2 + 1) * E + base + i * CH, 8)
                pltpu.sync_copy(edges_hbm.at[pl.ds(off_s, CH)], sidx_v)
                pltpu.sync_copy(edges_hbm.at[pl.ds(off_d, CH)], didx_v)
                pltpu.async_copy(hs_hbm.at[r].at[sidx_v], rows_v, sem).wait()
                pltpu.sync_copy(rows_v, acc_sp.at[didx_v], add=True)
                return carry

            lax.fori_loop(0, NCH, body, 0)
            plsc.subcore_barrier()

            @pl.when(s == 0)
            def _():
                pltpu.sync_copy(acc_sp, out_hbm.at[c, r])
            plsc.subcore_barrier()

    return agg


_agg_hid = _make_agg(HID)


@functools.partial(
    pl.kernel, mesh=_mesh,
    out_type=jax.ShapeDtypeStruct((R, B, HID), jnp.float32),
    scratch_types=[
        pltpu.VMEM((BPW, HID), jnp.float32),
        pltpu.VMEM((BPW,), jnp.int32),
        pltpu.SemaphoreType.DMA,
    ],
)
def _batch_gather(z_hbm, bidx_hbm, out_hbm, rows_v, idx_v, sem):
    c = lax.axis_index("c")
    s = lax.axis_index("s")
    w = c * NS + s
    off = pl.multiple_of(w * BPW, 8)
    pltpu.sync_copy(bidx_hbm.at[pl.ds(off, BPW)], idx_v)
    for r in range(R):
        pltpu.async_copy(z_hbm.at[r].at[idx_v], rows_v, sem).wait()
        pltpu.sync_copy(rows_v, out_hbm.at[r, pl.ds(off, BPW)])


_BNROWS = 2000


def _dinv_of(deg_ref, r_axis=False):
    deg = deg_ref[0, 0, :, 0] + deg_ref[1, 0, :, 0] + 1.0
    return lax.rsqrt(deg)


def _mm1_body(feat_ref, w_ref, deg_ref, hs_ref):
    dinv = _dinv_of(deg_ref)
    h = jnp.dot(feat_ref[...], w_ref[0], preferred_element_type=jnp.float32)
    hs_ref[0] = h * dinv[:, None]


def _mm2_body(acc_ref, hs1_ref, deg_ref, w2_ref, b1_ref, g1_ref, bt1_ref,
              hs2_ref):
    dinv = _dinv_of(deg_ref)
    pre = dinv[:, None] * (acc_ref[0, 0] + acc_ref[1, 0] + hs1_ref[0]) + b1_ref[0]
    x1 = jnp.maximum(pre * (BN_SCALE * g1_ref[0]) + bt1_ref[0], 0.0)
    hs2_ref[0] = jnp.dot(x1, w2_ref[0],
                         preferred_element_type=jnp.float32) * dinv[:, None]


def _final_pre_body(acc_ref, hs2_ref, deg_ref, b2_ref, z_ref):
    dinv = _dinv_of(deg_ref)
    z_ref[0] = dinv[:, None] * (acc_ref[0, 0] + acc_ref[1, 0] + hs2_ref[0]) + b2_ref[0]


def _logsoftmax_body(z_ref, g2_ref, bt2_ref, out_ref):
    y = jnp.maximum(z_ref[0, :, :OUT] * (BN_SCALE * g2_ref[0]) + bt2_ref[0], 0.0)
    m = jnp.max(y, axis=-1, keepdims=True)
    e = jnp.exp(y - m)
    out_ref[0] = (y - m) - jnp.log(jnp.sum(e, axis=-1, keepdims=True))


def _tc_hs1(features, W1, deg8):
    grid = (R, N // _BNROWS)
    return pl.pallas_call(
        _mm1_body,
        grid=grid,
        in_specs=[
            pl.BlockSpec((_BNROWS, F_IN), lambda r, j: (j, 0)),
            pl.BlockSpec((1, F_IN, HID), lambda r, j: (r, 0, 0)),
            pl.BlockSpec((NC, 1, _BNROWS, HID), lambda r, j: (0, r, j, 0)),
        ],
        out_specs=pl.BlockSpec((1, _BNROWS, HID), lambda r, j: (r, j, 0)),
        out_shape=jax.ShapeDtypeStruct((R, N, HID), jnp.float32),
    )(features, W1, deg8)


def _tc_hs2(acc1, hs1, deg8, W2, b1, g1, beta1):
    grid = (R, N // _BNROWS)
    return pl.pallas_call(
        _mm2_body,
        grid=grid,
        in_specs=[
            pl.BlockSpec((NC, 1, _BNROWS, HID), lambda r, j: (0, r, j, 0)),
            pl.BlockSpec((1, _BNROWS, HID), lambda r, j: (r, j, 0)),
            pl.BlockSpec((NC, 1, _BNROWS, HID), lambda r, j: (0, r, j, 0)),
            pl.BlockSpec((1, HID, HID), lambda r, j: (r, 0, 0)),
            pl.BlockSpec((1, 1, HID), lambda r, j: (r, 0, 0)),
            pl.BlockSpec((1, 1, HID), lambda r, j: (r, 0, 0)),
            pl.BlockSpec((1, 1, HID), lambda r, j: (r, 0, 0)),
        ],
        out_specs=pl.BlockSpec((1, _BNROWS, HID), lambda r, j: (r, j, 0)),
        out_shape=jax.ShapeDtypeStruct((R, N, HID), jnp.float32),
    )(acc1, hs1, deg8, W2, b1, g1, beta1)


def _tc_final_pre(acc2, hs2, deg8, b2):
    grid = (R, N // _BNROWS)
    return pl.pallas_call(
        _final_pre_body,
        grid=grid,
        in_specs=[
            pl.BlockSpec((NC, 1, _BNROWS, HID), lambda r, j: (0, r, j, 0)),
            pl.BlockSpec((1, _BNROWS, HID), lambda r, j: (r, j, 0)),
            pl.BlockSpec((NC, 1, _BNROWS, HID), lambda r, j: (0, r, j, 0)),
            pl.BlockSpec((1, 1, HID), lambda r, j: (r, 0, 0)),
        ],
        out_specs=pl.BlockSpec((1, _BNROWS, HID), lambda r, j: (r, j, 0)),
        out_shape=jax.ShapeDtypeStruct((R, N, HID), jnp.float32),
    )(acc2, hs2, deg8, b2)


def _tc_logsoftmax(gath, g2, beta2):
    return pl.pallas_call(
        _logsoftmax_body,
        grid=(R,),
        in_specs=[
            pl.BlockSpec((1, B, HID), lambda r: (r, 0, 0)),
            pl.BlockSpec((1, 1, OUT), lambda r: (r, 0, 0)),
            pl.BlockSpec((1, 1, OUT), lambda r: (r, 0, 0)),
        ],
        out_specs=pl.BlockSpec((1, B, OUT), lambda r: (r, 0, 0)),
        out_shape=jax.ShapeDtypeStruct((R, B, OUT), jnp.float32),
    )(gath, g2, beta2)


def kernel(features, multi_r_data, batch_nodes, W1, b1, g1, beta1,
           W2, b2, g2, beta2):
    ones_rows = jnp.ones((CH, HID), jnp.float32)
    zeros_nh = jnp.zeros((N, HID), jnp.float32)
    b1r = b1.reshape(R, 1, HID)
    g1r = g1.reshape(R, 1, HID)
    beta1r = beta1.reshape(R, 1, HID)
    W2p = jnp.pad(W2, ((0, 0), (0, 0), (0, HID - OUT)))
    b2r = jnp.pad(b2, ((0, 0), (0, HID - OUT))).reshape(R, 1, HID)
    g2r = g2.reshape(R, 1, OUT)
    beta2r = beta2.reshape(R, 1, OUT)

    edges_flat = multi_r_data.reshape(R * 2 * E)
    deg8 = _deg_kernel(edges_flat, ones_rows, zeros_nh)
    hs1 = _tc_hs1(features, W1, deg8)
    acc1 = _agg_hid(hs1, edges_flat, zeros_nh)
    hs2 = _tc_hs2(acc1, hs1, deg8, W2p, b1r, g1r, beta1r)
    acc2 = _agg_hid(hs2, edges_flat, zeros_nh)
    z = _tc_final_pre(acc2, hs2, deg8, b2r)
    gath = _batch_gather(z, batch_nodes)
    out = _tc_logsoftmax(gath, g2r, beta2r)
    return jnp.transpose(out, (1, 0, 2)).reshape(B, R * OUT)


# speedup vs baseline: 22.1502x; 2.1284x over previous
import functools
import math

import jax
import jax.numpy as jnp
from jax import lax
from jax.experimental import pallas as pl
from jax.experimental.pallas import tpu as pltpu
from jax.experimental.pallas import tpu_sc as plsc

N = 10000
E = 320000
R = 3
F_IN = 128
HID = 128
OUT = 64
B = 1024
EPS = 1e-5
BN_SCALE = 1.0 / math.sqrt(1.0 + EPS)

NC = 2
NS = 16
NW = NC * NS
CH = 128
EPW = E // NW
NCH = 80
EPW_PAD = NCH * CH
NP = N + (EPW_PAD - EPW)
RPT = NP // NS
BPW = B // NW

_mesh = plsc.VectorSubcoreMesh(core_axis_name="c", subcore_axis_name="s")


@functools.partial(
    pl.kernel, mesh=_mesh,
    out_type=jax.ShapeDtypeStruct((NC, R, NP, HID), jnp.float32),
    scratch_types=[
        pltpu.VMEM_SHARED((NP, HID), jnp.float32),
        pltpu.VMEM((CH, HID), jnp.float32),
        pltpu.VMEM((NCH, CH), jnp.int32),
    ],
)
def _deg_kernel(dstp_hbm, ones_hbm, zeros_hbm, out_hbm, deg_sp, ones_v,
                didx_v):
    c = lax.axis_index("c")
    s = lax.axis_index("s")
    w = c * NS + s
    pltpu.sync_copy(ones_hbm, ones_v)
    for r in range(R):
        pltpu.sync_copy(zeros_hbm.at[pl.ds(RPT * s, RPT)],
                        deg_sp.at[pl.ds(RPT * s, RPT)])
        pltpu.sync_copy(dstp_hbm.at[r, w], didx_v)
        plsc.subcore_barrier()

        def body(i, carry):
            pltpu.sync_copy(ones_v, deg_sp.at[didx_v.at[i]], add=True)
            return carry

        lax.fori_loop(0, NCH, body, 0)
        plsc.subcore_barrier()
        pltpu.sync_copy(deg_sp.at[pl.ds(RPT * s, RPT)],
                        out_hbm.at[c, r, pl.ds(RPT * s, RPT)])
        plsc.subcore_barrier()


@functools.partial(
    pl.kernel, mesh=_mesh,
    out_type=jax.ShapeDtypeStruct((NC, R, NP, HID), jnp.float32),
    scratch_types=[
        pltpu.VMEM_SHARED((NP, HID), jnp.float32),
        pltpu.VMEM((CH, HID), jnp.float32),
        pltpu.VMEM((CH, HID), jnp.float32),
        pltpu.VMEM((NCH // 2, CH), jnp.int32),
        pltpu.VMEM((NCH // 2, CH), jnp.int32),
        pltpu.SemaphoreType.DMA,
        pltpu.SemaphoreType.DMA,
    ],
)
def _agg_kernel(hs_hbm, srcp_hbm, dstp_hbm, zeros_hbm, out_hbm, acc_sp,
                rows0, rows1, sidx_v, didx_v, sem0, sem1):
    c = lax.axis_index("c")
    s = lax.axis_index("s")
    w = c * NS + s
    HCH = NCH // 2
    for r in range(R):
        pltpu.sync_copy(zeros_hbm.at[pl.ds(RPT * s, RPT)],
                        acc_sp.at[pl.ds(RPT * s, RPT)])
        plsc.subcore_barrier()
        for h in range(2):
            pltpu.sync_copy(srcp_hbm.at[r, w, pl.ds(HCH * h, HCH)], sidx_v)
            pltpu.sync_copy(dstp_hbm.at[r, w, pl.ds(HCH * h, HCH)], didx_v)
            pltpu.async_copy(hs_hbm.at[r].at[sidx_v.at[0]], rows0, sem0)
            pltpu.async_copy(hs_hbm.at[r].at[sidx_v.at[1]], rows1, sem1)

            def body(i, carry):
                i0 = 2 * i
                pltpu.make_async_copy(hs_hbm.at[r, pl.ds(0, CH)], rows0,
                                      sem0).wait()
                pltpu.sync_copy(rows0, acc_sp.at[didx_v.at[i0]], add=True)
                nxt0 = jnp.minimum(i0 + 2, HCH - 1)
                pltpu.async_copy(hs_hbm.at[r].at[sidx_v.at[nxt0]], rows0,
                                 sem0)
                pltpu.make_async_copy(hs_hbm.at[r, pl.ds(0, CH)], rows1,
                                      sem1).wait()
                pltpu.sync_copy(rows1, acc_sp.at[didx_v.at[i0 + 1]],
                                add=True)
                nxt1 = jnp.minimum(i0 + 3, HCH - 1)
                pltpu.async_copy(hs_hbm.at[r].at[sidx_v.at[nxt1]], rows1,
                                 sem1)
                return carry

            lax.fori_loop(0, HCH // 2, body, 0)
            pltpu.make_async_copy(hs_hbm.at[r, pl.ds(0, CH)], rows0,
                                  sem0).wait()
            pltpu.make_async_copy(hs_hbm.at[r, pl.ds(0, CH)], rows1,
                                  sem1).wait()
        plsc.subcore_barrier()
        pltpu.sync_copy(acc_sp.at[pl.ds(RPT * s, RPT)],
                        out_hbm.at[c, r, pl.ds(RPT * s, RPT)])
        plsc.subcore_barrier()


@functools.partial(
    pl.kernel, mesh=_mesh,
    out_type=jax.ShapeDtypeStruct((R, B, HID), jnp.float32),
    scratch_types=[
        pltpu.VMEM((BPW, HID), jnp.float32),
        pltpu.VMEM((BPW,), jnp.int32),
        pltpu.SemaphoreType.DMA,
    ],
)
def _batch_gather(z_hbm, bidx_hbm, out_hbm, rows_v, idx_v, sem):
    c = lax.axis_index("c")
    s = lax.axis_index("s")
    w = c * NS + s
    off = pl.multiple_of(w * BPW, 8)
    pltpu.sync_copy(bidx_hbm.at[pl.ds(off, BPW)], idx_v)
    for r in range(R):
        pltpu.async_copy(z_hbm.at[r].at[idx_v], rows_v, sem).wait()
        pltpu.sync_copy(rows_v, out_hbm.at[r, pl.ds(off, BPW)])


_BNROWS = 2000


def _dinv_of(deg_ref):
    deg = deg_ref[0, 0, :, 0] + deg_ref[1, 0, :, 0] + 1.0
    return lax.rsqrt(deg)


def _mm1_body(feat_ref, w_ref, deg_ref, hs_ref):
    dinv = _dinv_of(deg_ref)
    h = jnp.dot(feat_ref[...], w_ref[0], preferred_element_type=jnp.float32)
    hs_ref[0] = h * dinv[:, None]


def _mm2_body(acc_ref, hs1_ref, deg_ref, w2_ref, b1_ref, g1_ref, bt1_ref,
              hs2_ref):
    dinv = _dinv_of(deg_ref)
    pre = dinv[:, None] * (acc_ref[0, 0] + acc_ref[1, 0] + hs1_ref[0]) + b1_ref[0]
    x1 = jnp.maximum(pre * (BN_SCALE * g1_ref[0]) + bt1_ref[0], 0.0)
    hs2_ref[0] = jnp.dot(x1, w2_ref[0],
                         preferred_element_type=jnp.float32) * dinv[:, None]


def _final_pre_body(acc_ref, hs2_ref, deg_ref, b2_ref, z_ref):
    dinv = _dinv_of(deg_ref)
    z_ref[0] = dinv[:, None] * (acc_ref[0, 0] + acc_ref[1, 0] + hs2_ref[0]) + b2_ref[0]


def _logsoftmax_body(z_ref, g2_ref, bt2_ref, out_ref):
    y = jnp.maximum(z_ref[0, :, :OUT] * (BN_SCALE * g2_ref[0]) + bt2_ref[0], 0.0)
    m = jnp.max(y, axis=-1, keepdims=True)
    e = jnp.exp(y - m)
    out_ref[0] = (y - m) - jnp.log(jnp.sum(e, axis=-1, keepdims=True))


def _tc_hs1(features, W1, deg):
    grid = (R, N // _BNROWS)
    return pl.pallas_call(
        _mm1_body,
        grid=grid,
        in_specs=[
            pl.BlockSpec((_BNROWS, F_IN), lambda r, j: (j, 0)),
            pl.BlockSpec((1, F_IN, HID), lambda r, j: (r, 0, 0)),
            pl.BlockSpec((NC, 1, _BNROWS, HID), lambda r, j: (0, r, j, 0)),
        ],
        out_specs=pl.BlockSpec((1, _BNROWS, HID), lambda r, j: (r, j, 0)),
        out_shape=jax.ShapeDtypeStruct((R, N, HID), jnp.float32),
    )(features, W1, deg)


def _tc_hs2(acc1, hs1, deg, W2, b1, g1, beta1):
    grid = (R, N // _BNROWS)
    return pl.pallas_call(
        _mm2_body,
        grid=grid,
        in_specs=[
            pl.BlockSpec((NC, 1, _BNROWS, HID), lambda r, j: (0, r, j, 0)),
            pl.BlockSpec((1, _BNROWS, HID), lambda r, j: (r, j, 0)),
            pl.BlockSpec((NC, 1, _BNROWS, HID), lambda r, j: (0, r, j, 0)),
            pl.BlockSpec((1, HID, HID), lambda r, j: (r, 0, 0)),
            pl.BlockSpec((1, 1, HID), lambda r, j: (r, 0, 0)),
            pl.BlockSpec((1, 1, HID), lambda r, j: (r, 0, 0)),
            pl.BlockSpec((1, 1, HID), lambda r, j: (r, 0, 0)),
        ],
        out_specs=pl.BlockSpec((1, _BNROWS, HID), lambda r, j: (r, j, 0)),
        out_shape=jax.ShapeDtypeStruct((R, N, HID), jnp.float32),
    )(acc1, hs1, deg, W2, b1, g1, beta1)


def _tc_final_pre(acc2, hs2, deg, b2):
    grid = (R, N // _BNROWS)
    return pl.pallas_call(
        _final_pre_body,
        grid=grid,
        in_specs=[
            pl.BlockSpec((NC, 1, _BNROWS, HID), lambda r, j: (0, r, j, 0)),
            pl.BlockSpec((1, _BNROWS, HID), lambda r, j: (r, j, 0)),
            pl.BlockSpec((NC, 1, _BNROWS, HID), lambda r, j: (0, r, j, 0)),
            pl.BlockSpec((1, 1, HID), lambda r, j: (r, 0, 0)),
        ],
        out_specs=pl.BlockSpec((1, _BNROWS, HID), lambda r, j: (r, j, 0)),
        out_shape=jax.ShapeDtypeStruct((R, N, HID), jnp.float32),
    )(acc2, hs2, deg, b2)


def _tc_logsoftmax(gath, g2, beta2):
    return pl.pallas_call(
        _logsoftmax_body,
        grid=(R,),
        in_specs=[
            pl.BlockSpec((1, B, HID), lambda r: (r, 0, 0)),
            pl.BlockSpec((1, 1, OUT), lambda r: (r, 0, 0)),
            pl.BlockSpec((1, 1, OUT), lambda r: (r, 0, 0)),
        ],
        out_specs=pl.BlockSpec((1, B, OUT), lambda r: (r, 0, 0)),
        out_shape=jax.ShapeDtypeStruct((R, B, OUT), jnp.float32),
    )(gath, g2, beta2)


def _pad_edges(multi_r_data):
    src = multi_r_data[:, 0, :].reshape(R, NW, EPW)
    dst = multi_r_data[:, 1, :].reshape(R, NW, EPW)
    padn = EPW_PAD - EPW
    pad_src = jnp.broadcast_to(
        (jnp.arange(padn, dtype=jnp.int32) * 41) % N, (R, NW, padn))
    pad_dst = jnp.broadcast_to(
        N + jnp.arange(padn, dtype=jnp.int32), (R, NW, padn))
    srcp = jnp.concatenate([src, pad_src], axis=2).reshape(R, NW, NCH, CH)
    dstp = jnp.concatenate([dst, pad_dst], axis=2).reshape(R, NW, NCH, CH)
    return srcp, dstp


def kernel(features, multi_r_data, batch_nodes, W1, b1, g1, beta1,
           W2, b2, g2, beta2):
    ones_rows = jnp.ones((CH, HID), jnp.float32)
    zeros_np = jnp.zeros((NP, HID), jnp.float32)
    b1r = b1.reshape(R, 1, HID)
    g1r = g1.reshape(R, 1, HID)
    beta1r = beta1.reshape(R, 1, HID)
    W2p = jnp.pad(W2, ((0, 0), (0, 0), (0, HID - OUT)))
    b2r = jnp.pad(b2, ((0, 0), (0, HID - OUT))).reshape(R, 1, HID)
    g2r = g2.reshape(R, 1, OUT)
    beta2r = beta2.reshape(R, 1, OUT)

    srcp, dstp = _pad_edges(multi_r_data)
    deg = _deg_kernel(dstp, ones_rows, zeros_np)
    hs1 = _tc_hs1(features, W1, deg)
    acc1 = _agg_kernel(hs1, srcp, dstp, zeros_np)
    hs2 = _tc_hs2(acc1, hs1, deg, W2p, b1r, g1r, beta1r)
    acc2 = _agg_kernel(hs2, srcp, dstp, zeros_np)
    z = _tc_final_pre(acc2, hs2, deg, b2r)
    gath = _batch_gather(z, batch_nodes)
    out = _tc_logsoftmax(gath, g2r, beta2r)
    return jnp.transpose(out, (1, 0, 2)).reshape(B, R * OUT)


# speedup vs baseline: 24.6854x; 1.1145x over previous
import functools
import math

import jax
import jax.numpy as jnp
from jax import lax
from jax.experimental import pallas as pl
from jax.experimental.pallas import tpu as pltpu
from jax.experimental.pallas import tpu_sc as plsc

N = 10000
E = 320000
R = 3
F_IN = 128
HID = 128
OUT = 64
B = 1024
EPS = 1e-5
BN_SCALE = 1.0 / math.sqrt(1.0 + EPS)

NC = 2
NS = 16
NW = NC * NS
CH = 128
EPW = E // NW
NCH = 80
EPW_PAD = NCH * CH
NP = N + (EPW_PAD - EPW)
RPT = NP // NS
BPW = B // NW

_mesh = plsc.VectorSubcoreMesh(core_axis_name="c", subcore_axis_name="s")


NPLN = 8
NROW = NP // 128


@functools.partial(
    pl.kernel, mesh=_mesh,
    compiler_params=pltpu.CompilerParams(needs_layout_passes=False),
    out_type=jax.ShapeDtypeStruct((NC, NS, R, NPLN * NROW, 128), jnp.float32),
    scratch_types=[
        pltpu.VMEM((NPLN * NROW, 128), jnp.float32),
        pltpu.VMEM((NCH, CH), jnp.int32),
    ],
)
def _deg_kernel(dstp_hbm, zeros_hbm, out_hbm, hist_v, didx_v):
    c = lax.axis_index("c")
    s = lax.axis_index("s")
    w = c * NS + s
    lane = lax.iota(jnp.int32, 16)
    plane_row = (lane % NPLN) * NROW
    m_lo = lane < 8
    m_hi = lane >= 8
    ones16 = jnp.ones((16,), jnp.float32)
    for r in range(R):
        pltpu.sync_copy(zeros_hbm.at[pl.ds(0, NPLN * NROW)], hist_v)
        pltpu.sync_copy(dstp_hbm.at[r, w], didx_v)

        def body(i, carry):
            for j in range(CH // 16):
                dst = didx_v[i, pl.ds(16 * j, 16)]
                hrow = plane_row + lax.shift_right_logical(dst, 7)
                hcol = lax.bitwise_and(dst, 127)
                plsc.addupdate_scatter(hist_v, [hrow, hcol], ones16,
                                       mask=m_lo)
                plsc.addupdate_scatter(hist_v, [hrow, hcol], ones16,
                                       mask=m_hi)
            return carry

        lax.fori_loop(0, NCH, body, 0)
        pltpu.sync_copy(hist_v, out_hbm.at[c, s, r])


@functools.partial(
    pl.kernel, mesh=_mesh,
    out_type=jax.ShapeDtypeStruct((NC, R, NP, HID), jnp.float32),
    scratch_types=[
        pltpu.VMEM_SHARED((NP, HID), jnp.float32),
        pltpu.VMEM((CH, HID), jnp.float32),
        pltpu.VMEM((CH, HID), jnp.float32),
        pltpu.VMEM((NCH // 2, CH), jnp.int32),
        pltpu.VMEM((NCH // 2, CH), jnp.int32),
        pltpu.SemaphoreType.DMA,
        pltpu.SemaphoreType.DMA,
    ],
)
def _agg_kernel(hs_hbm, srcp_hbm, dstp_hbm, zeros_hbm, out_hbm, acc_sp,
                rows0, rows1, sidx_v, didx_v, sem0, sem1):
    c = lax.axis_index("c")
    s = lax.axis_index("s")
    w = c * NS + s
    HCH = NCH // 2
    for r in range(R):
        pltpu.sync_copy(zeros_hbm.at[pl.ds(RPT * s, RPT)],
                        acc_sp.at[pl.ds(RPT * s, RPT)])
        plsc.subcore_barrier()
        for h in range(2):
            pltpu.sync_copy(srcp_hbm.at[r, w, pl.ds(HCH * h, HCH)], sidx_v)
            pltpu.sync_copy(dstp_hbm.at[r, w, pl.ds(HCH * h, HCH)], didx_v)
            pltpu.async_copy(hs_hbm.at[r].at[sidx_v.at[0]], rows0, sem0)
            pltpu.async_copy(hs_hbm.at[r].at[sidx_v.at[1]], rows1, sem1)

            def body(i, carry):
                i0 = 2 * i
                pltpu.make_async_copy(hs_hbm.at[r, pl.ds(0, CH)], rows0,
                                      sem0).wait()
                pltpu.sync_copy(rows0, acc_sp.at[didx_v.at[i0]], add=True)
                nxt0 = jnp.minimum(i0 + 2, HCH - 1)
                pltpu.async_copy(hs_hbm.at[r].at[sidx_v.at[nxt0]], rows0,
                                 sem0)
                pltpu.make_async_copy(hs_hbm.at[r, pl.ds(0, CH)], rows1,
                                      sem1).wait()
                pltpu.sync_copy(rows1, acc_sp.at[didx_v.at[i0 + 1]],
                                add=True)
                nxt1 = jnp.minimum(i0 + 3, HCH - 1)
                pltpu.async_copy(hs_hbm.at[r].at[sidx_v.at[nxt1]], rows1,
                                 sem1)
                return carry

            lax.fori_loop(0, HCH // 2, body, 0)
            pltpu.make_async_copy(hs_hbm.at[r, pl.ds(0, CH)], rows0,
                                  sem0).wait()
            pltpu.make_async_copy(hs_hbm.at[r, pl.ds(0, CH)], rows1,
                                  sem1).wait()
        plsc.subcore_barrier()
        pltpu.sync_copy(acc_sp.at[pl.ds(RPT * s, RPT)],
                        out_hbm.at[c, r, pl.ds(RPT * s, RPT)])
        plsc.subcore_barrier()


@functools.partial(
    pl.kernel, mesh=_mesh,
    out_type=jax.ShapeDtypeStruct((R, B, HID), jnp.float32),
    scratch_types=[
        pltpu.VMEM((BPW, HID), jnp.float32),
        pltpu.VMEM((BPW,), jnp.int32),
        pltpu.SemaphoreType.DMA,
    ],
)
def _batch_gather(z_hbm, bidx_hbm, out_hbm, rows_v, idx_v, sem):
    c = lax.axis_index("c")
    s = lax.axis_index("s")
    w = c * NS + s
    off = pl.multiple_of(w * BPW, 8)
    pltpu.sync_copy(bidx_hbm.at[pl.ds(off, BPW)], idx_v)
    for r in range(R):
        pltpu.async_copy(z_hbm.at[r].at[idx_v], rows_v, sem).wait()
        pltpu.sync_copy(rows_v, out_hbm.at[r, pl.ds(off, BPW)])


_BNROWS = 2000


def _dinv_of(deg_ref):
    return lax.rsqrt(deg_ref[0, 0, 0] + 1.0)


def _deg_reduce_body(h_ref, o_ref):
    k = pl.program_id(2)
    part = jnp.sum(h_ref[...], axis=(0, 1))[0]

    @pl.when(k == 0)
    def _():
        o_ref[0] = part

    @pl.when(k != 0)
    def _():
        o_ref[0] = o_ref[0] + part


def _tc_deg_reduce(hist):
    return pl.pallas_call(
        _deg_reduce_body,
        grid=(R, NROW // 16, NPLN),
        in_specs=[
            pl.BlockSpec((NC, NS, 1, 16, 128),
                         lambda r, j, k: (0, 0, r, (NROW // 16) * k + j, 0)),
        ],
        out_specs=pl.BlockSpec((1, 16, 128), lambda r, j, k: (r, j, 0)),
        out_shape=jax.ShapeDtypeStruct((R, NROW, 128), jnp.float32),
    )(hist)


def _mm1_body(feat_ref, w_ref, deg_ref, hs_ref):
    dinv = _dinv_of(deg_ref)
    h = jnp.dot(feat_ref[...], w_ref[0], preferred_element_type=jnp.float32)
    hs_ref[0] = h * dinv[:, None]


def _mm2_body(acc_ref, hs1_ref, deg_ref, w2_ref, b1_ref, g1_ref, bt1_ref,
              hs2_ref):
    dinv = _dinv_of(deg_ref)
    pre = dinv[:, None] * (acc_ref[0, 0] + acc_ref[1, 0] + hs1_ref[0]) + b1_ref[0]
    x1 = jnp.maximum(pre * (BN_SCALE * g1_ref[0]) + bt1_ref[0], 0.0)
    hs2_ref[0] = jnp.dot(x1, w2_ref[0],
                         preferred_element_type=jnp.float32) * dinv[:, None]


def _final_pre_body(acc_ref, hs2_ref, deg_ref, b2_ref, z_ref):
    dinv = _dinv_of(deg_ref)
    z_ref[0] = dinv[:, None] * (acc_ref[0, 0] + acc_ref[1, 0] + hs2_ref[0]) + b2_ref[0]


def _logsoftmax_body(z_ref, g2_ref, bt2_ref, out_ref):
    y = jnp.maximum(z_ref[0, :, :OUT] * (BN_SCALE * g2_ref[0]) + bt2_ref[0], 0.0)
    m = jnp.max(y, axis=-1, keepdims=True)
    e = jnp.exp(y - m)
    out_ref[0] = (y - m) - jnp.log(jnp.sum(e, axis=-1, keepdims=True))


def _tc_hs1(features, W1, deg):
    grid = (R, N // _BNROWS)
    return pl.pallas_call(
        _mm1_body,
        grid=grid,
        in_specs=[
            pl.BlockSpec((_BNROWS, F_IN), lambda r, j: (j, 0)),
            pl.BlockSpec((1, F_IN, HID), lambda r, j: (r, 0, 0)),
            pl.BlockSpec((1, 1, 1, _BNROWS), lambda r, j: (r, j, 0, 0)),
        ],
        out_specs=pl.BlockSpec((1, _BNROWS, HID), lambda r, j: (r, j, 0)),
        out_shape=jax.ShapeDtypeStruct((R, N, HID), jnp.float32),
    )(features, W1, deg)


def _tc_hs2(acc1, hs1, deg, W2, b1, g1, beta1):
    grid = (R, N // _BNROWS)
    return pl.pallas_call(
        _mm2_body,
        grid=grid,
        in_specs=[
            pl.BlockSpec((NC, 1, _BNROWS, HID), lambda r, j: (0, r, j, 0)),
            pl.BlockSpec((1, _BNROWS, HID), lambda r, j: (r, j, 0)),
            pl.BlockSpec((1, 1, 1, _BNROWS), lambda r, j: (r, j, 0, 0)),
            pl.BlockSpec((1, HID, HID), lambda r, j: (r, 0, 0)),
            pl.BlockSpec((1, 1, HID), lambda r, j: (r, 0, 0)),
            pl.BlockSpec((1, 1, HID), lambda r, j: (r, 0, 0)),
            pl.BlockSpec((1, 1, HID), lambda r, j: (r, 0, 0)),
        ],
        out_specs=pl.BlockSpec((1, _BNROWS, HID), lambda r, j: (r, j, 0)),
        out_shape=jax.ShapeDtypeStruct((R, N, HID), jnp.float32),
    )(acc1, hs1, deg, W2, b1, g1, beta1)


def _tc_final_pre(acc2, hs2, deg, b2):
    grid = (R, N // _BNROWS)
    return pl.pallas_call(
        _final_pre_body,
        grid=grid,
        in_specs=[
            pl.BlockSpec((NC, 1, _BNROWS, HID), lambda r, j: (0, r, j, 0)),
            pl.BlockSpec((1, _BNROWS, HID), lambda r, j: (r, j, 0)),
            pl.BlockSpec((1, 1, 1, _BNROWS), lambda r, j: (r, j, 0, 0)),
            pl.BlockSpec((1, 1, HID), lambda r, j: (r, 0, 0)),
        ],
        out_specs=pl.BlockSpec((1, _BNROWS, HID), lambda r, j: (r, j, 0)),
        out_shape=jax.ShapeDtypeStruct((R, N, HID), jnp.float32),
    )(acc2, hs2, deg, b2)


def _tc_logsoftmax(gath, g2, beta2):
    return pl.pallas_call(
        _logsoftmax_body,
        grid=(R,),
        in_specs=[
            pl.BlockSpec((1, B, HID), lambda r: (r, 0, 0)),
            pl.BlockSpec((1, 1, OUT), lambda r: (r, 0, 0)),
            pl.BlockSpec((1, 1, OUT), lambda r: (r, 0, 0)),
        ],
        out_specs=pl.BlockSpec((1, B, OUT), lambda r: (r, 0, 0)),
        out_shape=jax.ShapeDtypeStruct((R, B, OUT), jnp.float32),
    )(gath, g2, beta2)


def _pad_edges(multi_r_data):
    src = multi_r_data[:, 0, :].reshape(R, NW, EPW)
    dst = multi_r_data[:, 1, :].reshape(R, NW, EPW)
    padn = EPW_PAD - EPW
    pad_src = jnp.broadcast_to(
        (jnp.arange(padn, dtype=jnp.int32) * 41) % N, (R, NW, padn))
    pad_dst = jnp.broadcast_to(
        N + jnp.arange(padn, dtype=jnp.int32), (R, NW, padn))
    srcp = jnp.concatenate([src, pad_src], axis=2).reshape(R, NW, NCH, CH)
    dstp = jnp.concatenate([dst, pad_dst], axis=2).reshape(R, NW, NCH, CH)
    return srcp, dstp


def kernel(features, multi_r_data, batch_nodes, W1, b1, g1, beta1,
           W2, b2, g2, beta2):
    zeros_np = jnp.zeros((NP, HID), jnp.float32)
    b1r = b1.reshape(R, 1, HID)
    g1r = g1.reshape(R, 1, HID)
    beta1r = beta1.reshape(R, 1, HID)
    W2p = jnp.pad(W2, ((0, 0), (0, 0), (0, HID - OUT)))
    b2r = jnp.pad(b2, ((0, 0), (0, HID - OUT))).reshape(R, 1, HID)
    g2r = g2.reshape(R, 1, OUT)
    beta2r = beta2.reshape(R, 1, OUT)

    srcp, dstp = _pad_edges(multi_r_data)
    hist = _deg_kernel(dstp, zeros_np)
    deg_flat = _tc_deg_reduce(hist)
    deg = deg_flat.reshape(R, NP)[:, :N].reshape(R, N // _BNROWS, 1, _BNROWS)
    hs1 = _tc_hs1(features, W1, deg)
    acc1 = _agg_kernel(hs1, srcp, dstp, zeros_np)
    hs2 = _tc_hs2(acc1, hs1, deg, W2p, b1r, g1r, beta1r)
    acc2 = _agg_kernel(hs2, srcp, dstp, zeros_np)
    z = _tc_final_pre(acc2, hs2, deg, b2r)
    gath = _batch_gather(z, batch_nodes)
    out = _tc_logsoftmax(gath, g2r, beta2r)
    return jnp.transpose(out, (1, 0, 2)).reshape(B, R * OUT)


# speedup vs baseline: 28.6419x; 1.1603x over previous
import functools
import math

import jax
import jax.numpy as jnp
from jax import lax
from jax.experimental import pallas as pl
from jax.experimental.pallas import tpu as pltpu
from jax.experimental.pallas import tpu_sc as plsc

N = 10000
E = 320000
R = 3
F_IN = 128
HID = 128
OUT = 64
B = 1024
EPS = 1e-5
BN_SCALE = 1.0 / math.sqrt(1.0 + EPS)

NC = 2
NS = 16
NW = NC * NS
CH = 128
EPW = E // NW
NCH = 80
EPW_PAD = NCH * CH
NP = N + (EPW_PAD - EPW)
RPT = NP // NS
BPW = B // NW

_mesh = plsc.VectorSubcoreMesh(core_axis_name="c", subcore_axis_name="s")


NPLN = 8
NROW = NP // 128


@functools.partial(
    pl.kernel, mesh=_mesh,
    compiler_params=pltpu.CompilerParams(needs_layout_passes=False),
    out_type=jax.ShapeDtypeStruct((NC, NS, R, NPLN * NROW, 128), jnp.float32),
    scratch_types=[
        pltpu.VMEM((NPLN * NROW, 128), jnp.float32),
        pltpu.VMEM((NCH, CH), jnp.int32),
    ],
)
def _deg_kernel(dstp_hbm, zeros_hbm, out_hbm, hist_v, didx_v):
    c = lax.axis_index("c")
    s = lax.axis_index("s")
    w = c * NS + s
    lane = lax.iota(jnp.int32, 16)
    plane_row = (lane % NPLN) * NROW
    m_lo = lane < 8
    m_hi = lane >= 8
    ones16 = jnp.ones((16,), jnp.float32)
    for r in range(R):
        pltpu.sync_copy(zeros_hbm.at[pl.ds(0, NPLN * NROW)], hist_v)
        pltpu.sync_copy(dstp_hbm.at[r, w], didx_v)

        def body(i, carry):
            for j in range(CH // 16):
                dst = didx_v[i, pl.ds(16 * j, 16)]
                hrow = plane_row + lax.shift_right_logical(dst, 7)
                hcol = lax.bitwise_and(dst, 127)
                plsc.addupdate_scatter(hist_v, [hrow, hcol], ones16,
                                       mask=m_lo)
                plsc.addupdate_scatter(hist_v, [hrow, hcol], ones16,
                                       mask=m_hi)
            return carry

        lax.fori_loop(0, NCH, body, 0)
        pltpu.sync_copy(hist_v, out_hbm.at[c, s, r])


@functools.partial(
    pl.kernel, mesh=_mesh,
    out_type=jax.ShapeDtypeStruct((NC, R, NP, HID), jnp.float32),
    scratch_types=[
        pltpu.VMEM_SHARED((NP, HID), jnp.float32),
        pltpu.VMEM((CH, HID), jnp.float32),
        pltpu.VMEM((CH, HID), jnp.float32),
        pltpu.VMEM((NCH // 2, CH), jnp.int32),
        pltpu.VMEM((NCH // 2, CH), jnp.int32),
        pltpu.SemaphoreType.DMA,
        pltpu.SemaphoreType.DMA,
    ],
)
def _agg_kernel(hs_hbm, srcp_hbm, dstp_hbm, zeros_hbm, out_hbm, acc_sp,
                rows0, rows1, sidx_v, didx_v, sem0, sem1):
    c = lax.axis_index("c")
    s = lax.axis_index("s")
    w = c * NS + s
    HCH = NCH // 2
    for r in range(R):
        pltpu.sync_copy(zeros_hbm.at[pl.ds(RPT * s, RPT)],
                        acc_sp.at[pl.ds(RPT * s, RPT)])
        plsc.subcore_barrier()
        for h in range(2):
            pltpu.sync_copy(srcp_hbm.at[r, w, pl.ds(HCH * h, HCH)], sidx_v)
            pltpu.sync_copy(dstp_hbm.at[r, w, pl.ds(HCH * h, HCH)], didx_v)
            pltpu.async_copy(hs_hbm.at[r].at[sidx_v.at[0]], rows0, sem0)
            pltpu.async_copy(hs_hbm.at[r].at[sidx_v.at[1]], rows1, sem1)

            def body(i, carry):
                i0 = 2 * i
                pltpu.make_async_copy(hs_hbm.at[r, pl.ds(0, CH)], rows0,
                                      sem0).wait()
                pltpu.sync_copy(rows0, acc_sp.at[didx_v.at[i0]], add=True)
                nxt0 = jnp.minimum(i0 + 2, HCH - 1)
                pltpu.async_copy(hs_hbm.at[r].at[sidx_v.at[nxt0]], rows0,
                                 sem0)
                pltpu.make_async_copy(hs_hbm.at[r, pl.ds(0, CH)], rows1,
                                      sem1).wait()
                pltpu.sync_copy(rows1, acc_sp.at[didx_v.at[i0 + 1]],
                                add=True)
                nxt1 = jnp.minimum(i0 + 3, HCH - 1)
                pltpu.async_copy(hs_hbm.at[r].at[sidx_v.at[nxt1]], rows1,
                                 sem1)
                return carry

            lax.fori_loop(0, HCH // 2, body, 0)
            pltpu.make_async_copy(hs_hbm.at[r, pl.ds(0, CH)], rows0,
                                  sem0).wait()
            pltpu.make_async_copy(hs_hbm.at[r, pl.ds(0, CH)], rows1,
                                  sem1).wait()
        plsc.subcore_barrier()
        pltpu.sync_copy(acc_sp.at[pl.ds(RPT * s, RPT)],
                        out_hbm.at[c, r, pl.ds(RPT * s, RPT)])
        plsc.subcore_barrier()


SEG = 8
NSEG = NCH // SEG
FCH = 64
CAP = SEG * CH + FCH


@functools.partial(
    pl.kernel, mesh=_mesh,
    compiler_params=pltpu.CompilerParams(needs_layout_passes=False),
    out_type=jax.ShapeDtypeStruct((NC, R, NP, HID), jnp.float32),
    scratch_types=[
        pltpu.VMEM_SHARED((NP, HID), jnp.float32),
        pltpu.VMEM((FCH, HID), jnp.float32),
        pltpu.VMEM((NROW, 128), jnp.int32),
        pltpu.VMEM((B,), jnp.int32),
        pltpu.VMEM((SEG, CH), jnp.int32),
        pltpu.VMEM((SEG, CH), jnp.int32),
        pltpu.VMEM((CAP,), jnp.int32),
        pltpu.VMEM((CAP,), jnp.int32),
        pltpu.VMEM((FCH,), jnp.int32),
        pltpu.SemaphoreType.DMA,
    ],
)
def _agg2_kernel(hs_hbm, srcp_hbm, dstp_hbm, zeros_hbm, zeros_i_hbm,
                 bidx_hbm, out_hbm, acc_sp, rows_v, inb_v, bb_v, sseg_v,
                 dseg_v, csrc_v, cdst_v, dchunk_v, sem):
    c = lax.axis_index("c")
    s = lax.axis_index("s")
    w = c * NS + s
    lane = lax.iota(jnp.int32, 16)
    ones_i = jnp.ones((16,), jnp.int32)
    dummy_src = (lane * 37) % N
    dummy_dst = N + lane
    pltpu.sync_copy(zeros_i_hbm, inb_v)
    pltpu.sync_copy(bidx_hbm, bb_v)

    def bset(i, carry):
        v = bb_v[pl.ds(16 * i, 16)]
        plsc.store_scatter(inb_v, [lax.shift_right_logical(v, 7),
                                   lax.bitwise_and(v, 127)], ones_i)
        return carry

    lax.fori_loop(0, B // 16, bset, 0)

    for r in range(R):
        pltpu.sync_copy(zeros_hbm.at[pl.ds(RPT * s, RPT)],
                        acc_sp.at[pl.ds(RPT * s, RPT)])
        plsc.subcore_barrier()

        def seg_body(g, carry):
            pltpu.sync_copy(srcp_hbm.at[r, w, pl.ds(SEG * g, SEG)], sseg_v)
            pltpu.sync_copy(dstp_hbm.at[r, w, pl.ds(SEG * g, SEG)], dseg_v)

            def chunk_body(i, cnt):
                for j in range(CH // 16):
                    dst = dseg_v[i, pl.ds(16 * j, 16)]
                    src = sseg_v[i, pl.ds(16 * j, 16)]
                    keep = plsc.load_gather(
                        inb_v, [lax.shift_right_logical(dst, 7),
                                lax.bitwise_and(dst, 127)]) > 0
                    plsc.store_compressed(cdst_v.at[pl.ds(cnt, 16)], dst,
                                          mask=keep)
                    plsc.store_compressed(csrc_v.at[pl.ds(cnt, 16)], src,
                                          mask=keep)
                    cnt = cnt + jnp.sum(keep.astype(jnp.int32))
                return cnt

            cnt = lax.fori_loop(0, SEG, chunk_body, 0)
            for k in range(FCH // 16):
                csrc_v[pl.ds(cnt + 16 * k, 16)] = dummy_src
                cdst_v[pl.ds(cnt + 16 * k, 16)] = dummy_dst

            def gs_body(q, carry2):
                pltpu.async_copy(
                    hs_hbm.at[r].at[csrc_v.at[pl.ds(FCH * q, FCH)]],
                    rows_v, sem).wait()
                for k in range(FCH // 16):
                    dchunk_v[pl.ds(16 * k, 16)] = cdst_v[
                        pl.ds(FCH * q + 16 * k, 16)]
                pltpu.sync_copy(rows_v, acc_sp.at[dchunk_v], add=True)
                return carry2

            nch = lax.shift_right_logical(cnt + (FCH - 1), 6)
            lax.fori_loop(0, nch, gs_body, 0)
            return carry

        lax.fori_loop(0, NSEG, seg_body, 0)
        plsc.subcore_barrier()
        pltpu.sync_copy(acc_sp.at[pl.ds(RPT * s, RPT)],
                        out_hbm.at[c, r, pl.ds(RPT * s, RPT)])
        plsc.subcore_barrier()


@functools.partial(
    pl.kernel, mesh=_mesh,
    out_type=jax.ShapeDtypeStruct((R, B, HID), jnp.float32),
    scratch_types=[
        pltpu.VMEM((BPW, HID), jnp.float32),
        pltpu.VMEM((BPW,), jnp.int32),
        pltpu.SemaphoreType.DMA,
    ],
)
def _batch_gather(z_hbm, bidx_hbm, out_hbm, rows_v, idx_v, sem):
    c = lax.axis_index("c")
    s = lax.axis_index("s")
    w = c * NS + s
    off = pl.multiple_of(w * BPW, 8)
    pltpu.sync_copy(bidx_hbm.at[pl.ds(off, BPW)], idx_v)
    for r in range(R):
        pltpu.async_copy(z_hbm.at[r].at[idx_v], rows_v, sem).wait()
        pltpu.sync_copy(rows_v, out_hbm.at[r, pl.ds(off, BPW)])


_BNROWS = 2000


def _dinv_of(deg_ref):
    return lax.rsqrt(deg_ref[0, 0, 0] + 1.0)


def _deg_reduce_body(h_ref, o_ref):
    k = pl.program_id(2)
    part = jnp.sum(h_ref[...], axis=(0, 1))[0]

    @pl.when(k == 0)
    def _():
        o_ref[0] = part

    @pl.when(k != 0)
    def _():
        o_ref[0] = o_ref[0] + part


def _tc_deg_reduce(hist):
    return pl.pallas_call(
        _deg_reduce_body,
        grid=(R, NROW // 16, NPLN),
        in_specs=[
            pl.BlockSpec((NC, NS, 1, 16, 128),
                         lambda r, j, k: (0, 0, r, (NROW // 16) * k + j, 0)),
        ],
        out_specs=pl.BlockSpec((1, 16, 128), lambda r, j, k: (r, j, 0)),
        out_shape=jax.ShapeDtypeStruct((R, NROW, 128), jnp.float32),
    )(hist)


def _mm1_body(feat_ref, w_ref, deg_ref, hs_ref):
    dinv = _dinv_of(deg_ref)
    h = jnp.dot(feat_ref[...], w_ref[0], preferred_element_type=jnp.float32)
    hs_ref[0] = h * dinv[:, None]


def _mm2_body(acc_ref, hs1_ref, deg_ref, w2_ref, b1_ref, g1_ref, bt1_ref,
              hs2_ref):
    dinv = _dinv_of(deg_ref)
    pre = dinv[:, None] * (acc_ref[0, 0] + acc_ref[1, 0] + hs1_ref[0]) + b1_ref[0]
    x1 = jnp.maximum(pre * (BN_SCALE * g1_ref[0]) + bt1_ref[0], 0.0)
    hs2_ref[0] = jnp.dot(x1, w2_ref[0],
                         preferred_element_type=jnp.float32) * dinv[:, None]


def _final_pre_body(acc_ref, hs2_ref, deg_ref, b2_ref, z_ref):
    dinv = _dinv_of(deg_ref)
    z_ref[0] = dinv[:, None] * (acc_ref[0, 0] + acc_ref[1, 0] + hs2_ref[0]) + b2_ref[0]


def _logsoftmax_body(z_ref, g2_ref, bt2_ref, out_ref):
    y = jnp.maximum(z_ref[0, :, :OUT] * (BN_SCALE * g2_ref[0]) + bt2_ref[0], 0.0)
    m = jnp.max(y, axis=-1, keepdims=True)
    e = jnp.exp(y - m)
    out_ref[0] = (y - m) - jnp.log(jnp.sum(e, axis=-1, keepdims=True))


def _tc_hs1(features, W1, deg):
    grid = (R, N // _BNROWS)
    return pl.pallas_call(
        _mm1_body,
        grid=grid,
        in_specs=[
            pl.BlockSpec((_BNROWS, F_IN), lambda r, j: (j, 0)),
            pl.BlockSpec((1, F_IN, HID), lambda r, j: (r, 0, 0)),
            pl.BlockSpec((1, 1, 1, _BNROWS), lambda r, j: (r, j, 0, 0)),
        ],
        out_specs=pl.BlockSpec((1, _BNROWS, HID), lambda r, j: (r, j, 0)),
        out_shape=jax.ShapeDtypeStruct((R, N, HID), jnp.float32),
    )(features, W1, deg)


def _tc_hs2(acc1, hs1, deg, W2, b1, g1, beta1):
    grid = (R, N // _BNROWS)
    return pl.pallas_call(
        _mm2_body,
        grid=grid,
        in_specs=[
            pl.BlockSpec((NC, 1, _BNROWS, HID), lambda r, j: (0, r, j, 0)),
            pl.BlockSpec((1, _BNROWS, HID), lambda r, j: (r, j, 0)),
            pl.BlockSpec((1, 1, 1, _BNROWS), lambda r, j: (r, j, 0, 0)),
            pl.BlockSpec((1, HID, HID), lambda r, j: (r, 0, 0)),
            pl.BlockSpec((1, 1, HID), lambda r, j: (r, 0, 0)),
            pl.BlockSpec((1, 1, HID), lambda r, j: (r, 0, 0)),
            pl.BlockSpec((1, 1, HID), lambda r, j: (r, 0, 0)),
        ],
        out_specs=pl.BlockSpec((1, _BNROWS, HID), lambda r, j: (r, j, 0)),
        out_shape=jax.ShapeDtypeStruct((R, N, HID), jnp.float32),
    )(acc1, hs1, deg, W2, b1, g1, beta1)


def _tc_final_pre(acc2, hs2, deg, b2):
    grid = (R, N // _BNROWS)
    return pl.pallas_call(
        _final_pre_body,
        grid=grid,
        in_specs=[
            pl.BlockSpec((NC, 1, _BNROWS, HID), lambda r, j: (0, r, j, 0)),
            pl.BlockSpec((1, _BNROWS, HID), lambda r, j: (r, j, 0)),
            pl.BlockSpec((1, 1, 1, _BNROWS), lambda r, j: (r, j, 0, 0)),
            pl.BlockSpec((1, 1, HID), lambda r, j: (r, 0, 0)),
        ],
        out_specs=pl.BlockSpec((1, _BNROWS, HID), lambda r, j: (r, j, 0)),
        out_shape=jax.ShapeDtypeStruct((R, N, HID), jnp.float32),
    )(acc2, hs2, deg, b2)


def _tc_logsoftmax(gath, g2, beta2):
    return pl.pallas_call(
        _logsoftmax_body,
        grid=(R,),
        in_specs=[
            pl.BlockSpec((1, B, HID), lambda r: (r, 0, 0)),
            pl.BlockSpec((1, 1, OUT), lambda r: (r, 0, 0)),
            pl.BlockSpec((1, 1, OUT), lambda r: (r, 0, 0)),
        ],
        out_specs=pl.BlockSpec((1, B, OUT), lambda r: (r, 0, 0)),
        out_shape=jax.ShapeDtypeStruct((R, B, OUT), jnp.float32),
    )(gath, g2, beta2)


def _pad_edges(multi_r_data):
    src = multi_r_data[:, 0, :].reshape(R, NW, EPW)
    dst = multi_r_data[:, 1, :].reshape(R, NW, EPW)
    padn = EPW_PAD - EPW
    pad_src = jnp.broadcast_to(
        (jnp.arange(padn, dtype=jnp.int32) * 41) % N, (R, NW, padn))
    pad_dst = jnp.broadcast_to(
        N + jnp.arange(padn, dtype=jnp.int32), (R, NW, padn))
    srcp = jnp.concatenate([src, pad_src], axis=2).reshape(R, NW, NCH, CH)
    dstp = jnp.concatenate([dst, pad_dst], axis=2).reshape(R, NW, NCH, CH)
    return srcp, dstp


def kernel(features, multi_r_data, batch_nodes, W1, b1, g1, beta1,
           W2, b2, g2, beta2):
    zeros_np = jnp.zeros((NP, HID), jnp.float32)
    b1r = b1.reshape(R, 1, HID)
    g1r = g1.reshape(R, 1, HID)
    beta1r = beta1.reshape(R, 1, HID)
    W2p = jnp.pad(W2, ((0, 0), (0, 0), (0, HID - OUT)))
    b2r = jnp.pad(b2, ((0, 0), (0, HID - OUT))).reshape(R, 1, HID)
    g2r = g2.reshape(R, 1, OUT)
    beta2r = beta2.reshape(R, 1, OUT)

    srcp, dstp = _pad_edges(multi_r_data)
    hist = _deg_kernel(dstp, zeros_np)
    deg_flat = _tc_deg_reduce(hist)
    deg = deg_flat.reshape(R, NP)[:, :N].reshape(R, N // _BNROWS, 1, _BNROWS)
    hs1 = _tc_hs1(features, W1, deg)
    acc1 = _agg_kernel(hs1, srcp, dstp, zeros_np)
    hs2 = _tc_hs2(acc1, hs1, deg, W2p, b1r, g1r, beta1r)
    zeros_i = jnp.zeros((NROW, 128), jnp.int32)
    acc2 = _agg2_kernel(hs2, srcp, dstp, zeros_np, zeros_i, batch_nodes)
    z = _tc_final_pre(acc2, hs2, deg, b2r)
    gath = _batch_gather(z, batch_nodes)
    out = _tc_logsoftmax(gath, g2r, beta2r)
    return jnp.transpose(out, (1, 0, 2)).reshape(B, R * OUT)


# speedup vs baseline: 29.6775x; 1.0362x over previous
import functools
import math

import jax
import jax.numpy as jnp
from jax import lax
from jax.experimental import pallas as pl
from jax.experimental.pallas import tpu as pltpu
from jax.experimental.pallas import tpu_sc as plsc

N = 10000
E = 320000
R = 3
F_IN = 128
HID = 128
OUT = 64
B = 1024
EPS = 1e-5
BN_SCALE = 1.0 / math.sqrt(1.0 + EPS)

NC = 2
NS = 16
NW = NC * NS
CH = 128
EPW = E // NW
NCH = 80
EPW_PAD = NCH * CH
NP = N + (EPW_PAD - EPW)
RPT = NP // NS
BPW = B // NW

_mesh = plsc.VectorSubcoreMesh(core_axis_name="c", subcore_axis_name="s")


NPLN = 8
NROW = NP // 128


@functools.partial(
    pl.kernel, mesh=_mesh,
    compiler_params=pltpu.CompilerParams(needs_layout_passes=False),
    out_type=jax.ShapeDtypeStruct((NC, NS, R, NPLN * NROW, 128), jnp.float32),
    scratch_types=[
        pltpu.VMEM((NPLN * NROW, 128), jnp.float32),
        pltpu.VMEM((NCH, CH), jnp.int32),
    ],
)
def _deg_kernel(dstp_hbm, zeros_hbm, out_hbm, hist_v, didx_v):
    c = lax.axis_index("c")
    s = lax.axis_index("s")
    w = c * NS + s
    lane = lax.iota(jnp.int32, 16)
    plane_row = (lane % NPLN) * NROW
    m_lo = lane < 8
    m_hi = lane >= 8
    ones16 = jnp.ones((16,), jnp.float32)
    for r in range(R):
        pltpu.sync_copy(zeros_hbm.at[pl.ds(0, NPLN * NROW)], hist_v)
        pltpu.sync_copy(dstp_hbm.at[r, w], didx_v)

        def body(i, carry):
            for j in range(CH // 16):
                dst = didx_v[i, pl.ds(16 * j, 16)]
                hrow = plane_row + lax.shift_right_logical(dst, 7)
                hcol = lax.bitwise_and(dst, 127)
                plsc.addupdate_scatter(hist_v, [hrow, hcol], ones16,
                                       mask=m_lo)
                plsc.addupdate_scatter(hist_v, [hrow, hcol], ones16,
                                       mask=m_hi)
            return carry

        lax.fori_loop(0, NCH, body, 0)
        pltpu.sync_copy(hist_v, out_hbm.at[c, s, r])


@functools.partial(
    pl.kernel, mesh=_mesh,
    out_type=jax.ShapeDtypeStruct((NC, R, NP, HID), jnp.float32),
    scratch_types=[
        pltpu.VMEM_SHARED((NP, HID), jnp.float32),
        pltpu.VMEM((CH, HID), jnp.float32),
        pltpu.VMEM((CH, HID), jnp.float32),
        pltpu.VMEM((NCH // 2, CH), jnp.int32),
        pltpu.VMEM((NCH // 2, CH), jnp.int32),
        pltpu.SemaphoreType.DMA,
        pltpu.SemaphoreType.DMA,
    ],
)
def _agg_kernel(hs_hbm, srcp_hbm, dstp_hbm, zeros_hbm, out_hbm, acc_sp,
                rows0, rows1, sidx_v, didx_v, sem0, sem1):
    c = lax.axis_index("c")
    s = lax.axis_index("s")
    w = c * NS + s
    HCH = NCH // 2
    for r in range(R):
        pltpu.sync_copy(zeros_hbm.at[pl.ds(RPT * s, RPT)],
                        acc_sp.at[pl.ds(RPT * s, RPT)])
        plsc.subcore_barrier()
        for h in range(2):
            pltpu.sync_copy(srcp_hbm.at[r, w, pl.ds(HCH * h, HCH)], sidx_v)
            pltpu.sync_copy(dstp_hbm.at[r, w, pl.ds(HCH * h, HCH)], didx_v)
            pltpu.async_copy(hs_hbm.at[r].at[sidx_v.at[0]], rows0, sem0)
            pltpu.async_copy(hs_hbm.at[r].at[sidx_v.at[1]], rows1, sem1)

            def body(i, carry):
                i0 = 2 * i
                pltpu.make_async_copy(hs_hbm.at[r, pl.ds(0, CH)], rows0,
                                      sem0).wait()
                pltpu.sync_copy(rows0, acc_sp.at[didx_v.at[i0]], add=True)
                nxt0 = jnp.minimum(i0 + 2, HCH - 1)
                pltpu.async_copy(hs_hbm.at[r].at[sidx_v.at[nxt0]], rows0,
                                 sem0)
                pltpu.make_async_copy(hs_hbm.at[r, pl.ds(0, CH)], rows1,
                                      sem1).wait()
                pltpu.sync_copy(rows1, acc_sp.at[didx_v.at[i0 + 1]],
                                add=True)
                nxt1 = jnp.minimum(i0 + 3, HCH - 1)
                pltpu.async_copy(hs_hbm.at[r].at[sidx_v.at[nxt1]], rows1,
                                 sem1)
                return carry

            lax.fori_loop(0, HCH // 2, body, 0)
            pltpu.make_async_copy(hs_hbm.at[r, pl.ds(0, CH)], rows0,
                                  sem0).wait()
            pltpu.make_async_copy(hs_hbm.at[r, pl.ds(0, CH)], rows1,
                                  sem1).wait()
        plsc.subcore_barrier()
        pltpu.sync_copy(acc_sp.at[pl.ds(RPT * s, RPT)],
                        out_hbm.at[c, r, pl.ds(RPT * s, RPT)])
        plsc.subcore_barrier()


SEG = 16
NSEG = NCH // SEG
FCH = 64
CAP = SEG * CH + FCH


@functools.partial(
    pl.kernel, mesh=_mesh,
    compiler_params=pltpu.CompilerParams(needs_layout_passes=False),
    out_type=jax.ShapeDtypeStruct((NC, R, NP, HID), jnp.float32),
    scratch_types=[
        pltpu.VMEM_SHARED((NP, HID), jnp.float32),
        pltpu.VMEM((FCH, HID), jnp.float32),
        pltpu.VMEM((NROW, 128), jnp.int32),
        pltpu.VMEM((B,), jnp.int32),
        pltpu.VMEM((SEG, CH), jnp.int32),
        pltpu.VMEM((SEG, CH), jnp.int32),
        pltpu.VMEM((CAP,), jnp.int32),
        pltpu.VMEM((CAP,), jnp.int32),
        pltpu.VMEM((FCH,), jnp.int32),
        pltpu.SemaphoreType.DMA,
    ],
)
def _agg2_kernel(hs_hbm, srcp_hbm, dstp_hbm, zeros_hbm, zeros_i_hbm,
                 bidx_hbm, out_hbm, acc_sp, rows_v, inb_v, bb_v, sseg_v,
                 dseg_v, csrc_v, cdst_v, dchunk_v, sem):
    c = lax.axis_index("c")
    s = lax.axis_index("s")
    w = c * NS + s
    lane = lax.iota(jnp.int32, 16)
    ones_i = jnp.ones((16,), jnp.int32)
    dummy_src = (lane * 37) % N
    dummy_dst = N + lane
    pltpu.sync_copy(zeros_i_hbm, inb_v)
    pltpu.sync_copy(bidx_hbm, bb_v)

    def bset(i, carry):
        v = bb_v[pl.ds(16 * i, 16)]
        plsc.store_scatter(inb_v, [lax.shift_right_logical(v, 7),
                                   lax.bitwise_and(v, 127)], ones_i)
        return carry

    lax.fori_loop(0, B // 16, bset, 0)

    for r in range(R):
        pltpu.sync_copy(zeros_hbm.at[pl.ds(RPT * s, RPT)],
                        acc_sp.at[pl.ds(RPT * s, RPT)])
        plsc.subcore_barrier()

        def seg_body(g, carry):
            pltpu.sync_copy(srcp_hbm.at[r, w, pl.ds(SEG * g, SEG)], sseg_v)
            pltpu.sync_copy(dstp_hbm.at[r, w, pl.ds(SEG * g, SEG)], dseg_v)

            def chunk_body(i, cnt):
                for j in range(CH // 16):
                    dst = dseg_v[i, pl.ds(16 * j, 16)]
                    src = sseg_v[i, pl.ds(16 * j, 16)]
                    keep = plsc.load_gather(
                        inb_v, [lax.shift_right_logical(dst, 7),
                                lax.bitwise_and(dst, 127)]) > 0
                    plsc.store_compressed(cdst_v.at[pl.ds(cnt, 16)], dst,
                                          mask=keep)
                    plsc.store_compressed(csrc_v.at[pl.ds(cnt, 16)], src,
                                          mask=keep)
                    cnt = cnt + jnp.sum(keep.astype(jnp.int32))
                return cnt

            cnt = lax.fori_loop(0, SEG, chunk_body, 0)
            for k in range(FCH // 16):
                csrc_v[pl.ds(cnt + 16 * k, 16)] = dummy_src
                cdst_v[pl.ds(cnt + 16 * k, 16)] = dummy_dst

            def gs_body(q, carry2):
                pltpu.async_copy(
                    hs_hbm.at[r].at[csrc_v.at[pl.ds(FCH * q, FCH)]],
                    rows_v, sem).wait()
                for k in range(FCH // 16):
                    dchunk_v[pl.ds(16 * k, 16)] = cdst_v[
                        pl.ds(FCH * q + 16 * k, 16)]
                pltpu.sync_copy(rows_v, acc_sp.at[dchunk_v], add=True)
                return carry2

            nch = lax.shift_right_logical(cnt + (FCH - 1), 6)
            lax.fori_loop(0, nch, gs_body, 0)
            return carry

        lax.fori_loop(0, NSEG, seg_body, 0)
        plsc.subcore_barrier()
        pltpu.sync_copy(acc_sp.at[pl.ds(RPT * s, RPT)],
                        out_hbm.at[c, r, pl.ds(RPT * s, RPT)])
        plsc.subcore_barrier()


@functools.partial(
    pl.kernel, mesh=_mesh,
    out_type=jax.ShapeDtypeStruct((R, B, HID), jnp.float32),
    scratch_types=[
        pltpu.VMEM((BPW, HID), jnp.float32),
        pltpu.VMEM((BPW,), jnp.int32),
        pltpu.SemaphoreType.DMA,
    ],
)
def _batch_gather(z_hbm, bidx_hbm, out_hbm, rows_v, idx_v, sem):
    c = lax.axis_index("c")
    s = lax.axis_index("s")
    w = c * NS + s
    off = pl.multiple_of(w * BPW, 8)
    pltpu.sync_copy(bidx_hbm.at[pl.ds(off, BPW)], idx_v)
    for r in range(R):
        pltpu.async_copy(z_hbm.at[r].at[idx_v], rows_v, sem).wait()
        pltpu.sync_copy(rows_v, out_hbm.at[r, pl.ds(off, BPW)])


_BNROWS = 2000


def _dinv_of(deg_ref):
    return lax.rsqrt(deg_ref[0, 0, 0] + 1.0)


def _deg_reduce_body(h_ref, o_ref):
    k = pl.program_id(2)
    part = jnp.sum(h_ref[...], axis=(0, 1))[0]

    @pl.when(k == 0)
    def _():
        o_ref[0] = part

    @pl.when(k != 0)
    def _():
        o_ref[0] = o_ref[0] + part


def _tc_deg_reduce(hist):
    return pl.pallas_call(
        _deg_reduce_body,
        grid=(R, NROW // 16, NPLN),
        in_specs=[
            pl.BlockSpec((NC, NS, 1, 16, 128),
                         lambda r, j, k: (0, 0, r, (NROW // 16) * k + j, 0)),
        ],
        out_specs=pl.BlockSpec((1, 16, 128), lambda r, j, k: (r, j, 0)),
        out_shape=jax.ShapeDtypeStruct((R, NROW, 128), jnp.float32),
    )(hist)


def _mm1_body(feat_ref, w_ref, deg_ref, hs_ref):
    dinv = _dinv_of(deg_ref)
    h = jnp.dot(feat_ref[...], w_ref[0], preferred_element_type=jnp.float32)
    hs_ref[0] = h * dinv[:, None]


def _mm2_body(acc_ref, hs1_ref, deg_ref, w2_ref, b1_ref, g1_ref, bt1_ref,
              hs2_ref):
    dinv = _dinv_of(deg_ref)
    pre = dinv[:, None] * (acc_ref[0, 0] + acc_ref[1, 0] + hs1_ref[0]) + b1_ref[0]
    x1 = jnp.maximum(pre * (BN_SCALE * g1_ref[0]) + bt1_ref[0], 0.0)
    hs2_ref[0] = jnp.dot(x1, w2_ref[0],
                         preferred_element_type=jnp.float32) * dinv[:, None]


def _final_pre_body(acc_ref, hs2_ref, deg_ref, b2_ref, z_ref):
    dinv = _dinv_of(deg_ref)
    z_ref[0] = dinv[:, None] * (acc_ref[0, 0] + acc_ref[1, 0] + hs2_ref[0]) + b2_ref[0]


def _logsoftmax_body(z_ref, g2_ref, bt2_ref, out_ref):
    y = jnp.maximum(z_ref[0, :, :OUT] * (BN_SCALE * g2_ref[0]) + bt2_ref[0], 0.0)
    m = jnp.max(y, axis=-1, keepdims=True)
    e = jnp.exp(y - m)
    out_ref[0] = (y - m) - jnp.log(jnp.sum(e, axis=-1, keepdims=True))


def _tc_hs1(features, W1, deg):
    grid = (R, N // _BNROWS)
    return pl.pallas_call(
        _mm1_body,
        grid=grid,
        in_specs=[
            pl.BlockSpec((_BNROWS, F_IN), lambda r, j: (j, 0)),
            pl.BlockSpec((1, F_IN, HID), lambda r, j: (r, 0, 0)),
            pl.BlockSpec((1, 1, 1, _BNROWS), lambda r, j: (r, j, 0, 0)),
        ],
        out_specs=pl.BlockSpec((1, _BNROWS, HID), lambda r, j: (r, j, 0)),
        out_shape=jax.ShapeDtypeStruct((R, N, HID), jnp.float32),
    )(features, W1, deg)


def _tc_hs2(acc1, hs1, deg, W2, b1, g1, beta1):
    grid = (R, N // _BNROWS)
    return pl.pallas_call(
        _mm2_body,
        grid=grid,
        in_specs=[
            pl.BlockSpec((NC, 1, _BNROWS, HID), lambda r, j: (0, r, j, 0)),
            pl.BlockSpec((1, _BNROWS, HID), lambda r, j: (r, j, 0)),
            pl.BlockSpec((1, 1, 1, _BNROWS), lambda r, j: (r, j, 0, 0)),
            pl.BlockSpec((1, HID, HID), lambda r, j: (r, 0, 0)),
            pl.BlockSpec((1, 1, HID), lambda r, j: (r, 0, 0)),
            pl.BlockSpec((1, 1, HID), lambda r, j: (r, 0, 0)),
            pl.BlockSpec((1, 1, HID), lambda r, j: (r, 0, 0)),
        ],
        out_specs=pl.BlockSpec((1, _BNROWS, HID), lambda r, j: (r, j, 0)),
        out_shape=jax.ShapeDtypeStruct((R, N, HID), jnp.float32),
    )(acc1, hs1, deg, W2, b1, g1, beta1)


def _tc_final_pre(acc2, hs2, deg, b2):
    grid = (R, N // _BNROWS)
    return pl.pallas_call(
        _final_pre_body,
        grid=grid,
        in_specs=[
            pl.BlockSpec((NC, 1, _BNROWS, HID), lambda r, j: (0, r, j, 0)),
            pl.BlockSpec((1, _BNROWS, HID), lambda r, j: (r, j, 0)),
            pl.BlockSpec((1, 1, 1, _BNROWS), lambda r, j: (r, j, 0, 0)),
            pl.BlockSpec((1, 1, HID), lambda r, j: (r, 0, 0)),
        ],
        out_specs=pl.BlockSpec((1, _BNROWS, HID), lambda r, j: (r, j, 0)),
        out_shape=jax.ShapeDtypeStruct((R, N, HID), jnp.float32),
    )(acc2, hs2, deg, b2)


def _tc_logsoftmax(gath, g2, beta2):
    return pl.pallas_call(
        _logsoftmax_body,
        grid=(R,),
        in_specs=[
            pl.BlockSpec((1, B, HID), lambda r: (r, 0, 0)),
            pl.BlockSpec((1, 1, OUT), lambda r: (r, 0, 0)),
            pl.BlockSpec((1, 1, OUT), lambda r: (r, 0, 0)),
        ],
        out_specs=pl.BlockSpec((1, B, OUT), lambda r: (r, 0, 0)),
        out_shape=jax.ShapeDtypeStruct((R, B, OUT), jnp.float32),
    )(gath, g2, beta2)


def _pad_edges(multi_r_data):
    src = multi_r_data[:, 0, :].reshape(R, NW, EPW)
    dst = multi_r_data[:, 1, :].reshape(R, NW, EPW)
    padn = EPW_PAD - EPW
    pad_src = jnp.broadcast_to(
        (jnp.arange(padn, dtype=jnp.int32) * 41) % N, (R, NW, padn))
    pad_dst = jnp.broadcast_to(
        N + jnp.arange(padn, dtype=jnp.int32), (R, NW, padn))
    srcp = jnp.concatenate([src, pad_src], axis=2).reshape(R, NW, NCH, CH)
    dstp = jnp.concatenate([dst, pad_dst], axis=2).reshape(R, NW, NCH, CH)
    return srcp, dstp


def kernel(features, multi_r_data, batch_nodes, W1, b1, g1, beta1,
           W2, b2, g2, beta2):
    zeros_np = jnp.zeros((NP, HID), jnp.float32)
    b1r = b1.reshape(R, 1, HID)
    g1r = g1.reshape(R, 1, HID)
    beta1r = beta1.reshape(R, 1, HID)
    W2p = jnp.pad(W2, ((0, 0), (0, 0), (0, HID - OUT)))
    b2r = jnp.pad(b2, ((0, 0), (0, HID - OUT))).reshape(R, 1, HID)
    g2r = g2.reshape(R, 1, OUT)
    beta2r = beta2.reshape(R, 1, OUT)

    srcp, dstp = _pad_edges(multi_r_data)
    hist = _deg_kernel(dstp, zeros_np)
    deg_flat = _tc_deg_reduce(hist)
    deg = deg_flat.reshape(R, NP)[:, :N].reshape(R, N // _BNROWS, 1, _BNROWS)
    hs1 = _tc_hs1(features, W1, deg)
    acc1 = _agg_kernel(hs1, srcp, dstp, zeros_np)
    hs2 = _tc_hs2(acc1, hs1, deg, W2p, b1r, g1r, beta1r)
    zeros_i = jnp.zeros((NROW, 128), jnp.int32)
    acc2 = _agg2_kernel(hs2, srcp, dstp, zeros_np, zeros_i, batch_nodes)
    z = _tc_final_pre(acc2, hs2, deg, b2r)
    gath = _batch_gather(z, batch_nodes)
    out = _tc_logsoftmax(gath, g2r, beta2r)
    return jnp.transpose(out, (1, 0, 2)).reshape(B, R * OUT)


# speedup vs baseline: 30.3233x; 1.0218x over previous
import functools
import math

import jax
import jax.numpy as jnp
from jax import lax
from jax.experimental import pallas as pl
from jax.experimental.pallas import tpu as pltpu
from jax.experimental.pallas import tpu_sc as plsc

N = 10000
E = 320000
R = 3
F_IN = 128
HID = 128
OUT = 64
B = 1024
EPS = 1e-5
BN_SCALE = 1.0 / math.sqrt(1.0 + EPS)

NC = 2
NS = 16
NW = NC * NS
CH = 128
EPW = E // NW
NCH = 80
EPW_PAD = NCH * CH
NP = N + (EPW_PAD - EPW)
RPT = NP // NS
BPW = B // NW

_mesh = plsc.VectorSubcoreMesh(core_axis_name="c", subcore_axis_name="s")


NPLN = 8
NROW = NP // 128


@functools.partial(
    pl.kernel, mesh=_mesh,
    compiler_params=pltpu.CompilerParams(needs_layout_passes=False),
    out_type=jax.ShapeDtypeStruct((NC, NS, R, NPLN * NROW, 128), jnp.float32),
    scratch_types=[
        pltpu.VMEM((NPLN * NROW, 128), jnp.float32),
        pltpu.VMEM((NCH, CH), jnp.int32),
    ],
)
def _deg_kernel(dstp_hbm, zeros_hbm, out_hbm, hist_v, didx_v):
    c = lax.axis_index("c")
    s = lax.axis_index("s")
    w = c * NS + s
    lane = lax.iota(jnp.int32, 16)
    plane_row = (lane % NPLN) * NROW
    m_lo = lane < 8
    m_hi = lane >= 8
    ones16 = jnp.ones((16,), jnp.float32)
    for r in range(R):
        pltpu.sync_copy(zeros_hbm.at[pl.ds(0, NPLN * NROW)], hist_v)
        pltpu.sync_copy(dstp_hbm.at[r, w], didx_v)

        def body(i, carry):
            for j in range(CH // 16):
                dst = didx_v[i, pl.ds(16 * j, 16)]
                hrow = plane_row + lax.shift_right_logical(dst, 7)
                hcol = lax.bitwise_and(dst, 127)
                plsc.addupdate_scatter(hist_v, [hrow, hcol], ones16,
                                       mask=m_lo)
                plsc.addupdate_scatter(hist_v, [hrow, hcol], ones16,
                                       mask=m_hi)
            return carry

        lax.fori_loop(0, NCH, body, 0)
        pltpu.sync_copy(hist_v, out_hbm.at[c, s, r])


@functools.partial(
    pl.kernel, mesh=_mesh,
    out_type=jax.ShapeDtypeStruct((NC, R, NP, HID), jnp.float32),
    scratch_types=[
        pltpu.VMEM_SHARED((NP, HID), jnp.float32),
        pltpu.VMEM((CH, HID), jnp.float32),
        pltpu.VMEM((CH, HID), jnp.float32),
        pltpu.VMEM((NCH // 2, CH), jnp.int32),
        pltpu.VMEM((NCH // 2, CH), jnp.int32),
        pltpu.SemaphoreType.DMA,
        pltpu.SemaphoreType.DMA,
    ],
)
def _agg_kernel(hs_hbm, srcp_hbm, dstp_hbm, zeros_hbm, out_hbm, acc_sp,
                rows0, rows1, sidx_v, didx_v, sem0, sem1):
    c = lax.axis_index("c")
    s = lax.axis_index("s")
    w = c * NS + s
    HCH = NCH // 2
    for r in range(R):
        pltpu.sync_copy(zeros_hbm.at[pl.ds(RPT * s, RPT)],
                        acc_sp.at[pl.ds(RPT * s, RPT)])
        plsc.subcore_barrier()
        for h in range(2):
            pltpu.sync_copy(srcp_hbm.at[r, w, pl.ds(HCH * h, HCH)], sidx_v)
            pltpu.sync_copy(dstp_hbm.at[r, w, pl.ds(HCH * h, HCH)], didx_v)
            pltpu.async_copy(hs_hbm.at[r].at[sidx_v.at[0]], rows0, sem0)
            pltpu.async_copy(hs_hbm.at[r].at[sidx_v.at[1]], rows1, sem1)

            def body(i, carry):
                i0 = 2 * i
                pltpu.make_async_copy(hs_hbm.at[r, pl.ds(0, CH)], rows0,
                                      sem0).wait()
                pltpu.sync_copy(rows0, acc_sp.at[didx_v.at[i0]], add=True)
                nxt0 = jnp.minimum(i0 + 2, HCH - 1)
                pltpu.async_copy(hs_hbm.at[r].at[sidx_v.at[nxt0]], rows0,
                                 sem0)
                pltpu.make_async_copy(hs_hbm.at[r, pl.ds(0, CH)], rows1,
                                      sem1).wait()
                pltpu.sync_copy(rows1, acc_sp.at[didx_v.at[i0 + 1]],
                                add=True)
                nxt1 = jnp.minimum(i0 + 3, HCH - 1)
                pltpu.async_copy(hs_hbm.at[r].at[sidx_v.at[nxt1]], rows1,
                                 sem1)
                return carry

            lax.fori_loop(0, HCH // 2, body, 0)
            pltpu.make_async_copy(hs_hbm.at[r, pl.ds(0, CH)], rows0,
                                  sem0).wait()
            pltpu.make_async_copy(hs_hbm.at[r, pl.ds(0, CH)], rows1,
                                  sem1).wait()
        plsc.subcore_barrier()
        pltpu.sync_copy(acc_sp.at[pl.ds(RPT * s, RPT)],
                        out_hbm.at[c, r, pl.ds(RPT * s, RPT)])
        plsc.subcore_barrier()


SEG = 16
NSEG = NCH // SEG
FCH = 64
CAP = SEG * CH + FCH


@functools.partial(
    pl.kernel, mesh=_mesh,
    compiler_params=pltpu.CompilerParams(needs_layout_passes=False),
    out_type=jax.ShapeDtypeStruct((NC, R, NP, HID), jnp.float32),
    scratch_types=[
        pltpu.VMEM_SHARED((NP, HID), jnp.float32),
        pltpu.VMEM((FCH, HID), jnp.float32),
        pltpu.VMEM((NROW, 128), jnp.int32),
        pltpu.VMEM((B,), jnp.int32),
        pltpu.VMEM((SEG, CH), jnp.int32),
        pltpu.VMEM((SEG, CH), jnp.int32),
        pltpu.VMEM((CAP,), jnp.int32),
        pltpu.VMEM((CAP,), jnp.int32),
        pltpu.VMEM((FCH,), jnp.int32),
        pltpu.SemaphoreType.DMA,
    ],
)
def _agg2_kernel(hs_hbm, srcp_hbm, dstp_hbm, zeros_hbm, zeros_i_hbm,
                 bidx_hbm, out_hbm, acc_sp, rows_v, inb_v, bb_v, sseg_v,
                 dseg_v, csrc_v, cdst_v, dchunk_v, sem):
    c = lax.axis_index("c")
    s = lax.axis_index("s")
    w = c * NS + s
    lane = lax.iota(jnp.int32, 16)
    ones_i = jnp.ones((16,), jnp.int32)
    dummy_src = (lane * 37) % N
    dummy_dst = N + lane
    pltpu.sync_copy(zeros_i_hbm, inb_v)
    pltpu.sync_copy(bidx_hbm, bb_v)

    def bset(i, carry):
        v = bb_v[pl.ds(16 * i, 16)]
        plsc.store_scatter(inb_v, [lax.shift_right_logical(v, 7),
                                   lax.bitwise_and(v, 127)], ones_i)
        return carry

    lax.fori_loop(0, B // 16, bset, 0)

    for r in range(R):
        pltpu.sync_copy(zeros_hbm.at[pl.ds(RPT * s, RPT)],
                        acc_sp.at[pl.ds(RPT * s, RPT)])
        plsc.subcore_barrier()

        def seg_body(g, carry):
            pltpu.sync_copy(srcp_hbm.at[r, w, pl.ds(SEG * g, SEG)], sseg_v)
            pltpu.sync_copy(dstp_hbm.at[r, w, pl.ds(SEG * g, SEG)], dseg_v)

            def chunk_body(i, cnt):
                for j in range(CH // 16):
                    dst = dseg_v[i, pl.ds(16 * j, 16)]
                    src = sseg_v[i, pl.ds(16 * j, 16)]
                    keep = plsc.load_gather(
                        inb_v, [lax.shift_right_logical(dst, 7),
                                lax.bitwise_and(dst, 127)]) > 0
                    plsc.store_compressed(cdst_v.at[pl.ds(cnt, 16)], dst,
                                          mask=keep)
                    plsc.store_compressed(csrc_v.at[pl.ds(cnt, 16)], src,
                                          mask=keep)
                    cnt = cnt + jnp.sum(keep.astype(jnp.int32))
                return cnt

            cnt = lax.fori_loop(0, SEG, chunk_body, 0)
            for k in range(FCH // 16):
                csrc_v[pl.ds(cnt + 16 * k, 16)] = dummy_src
                cdst_v[pl.ds(cnt + 16 * k, 16)] = dummy_dst

            def gs_body(q, carry2):
                pltpu.async_copy(
                    hs_hbm.at[r].at[csrc_v.at[pl.ds(FCH * q, FCH)]],
                    rows_v, sem).wait()
                for k in range(FCH // 16):
                    dchunk_v[pl.ds(16 * k, 16)] = cdst_v[
                        pl.ds(FCH * q + 16 * k, 16)]
                pltpu.sync_copy(rows_v, acc_sp.at[dchunk_v], add=True)
                return carry2

            nch = lax.shift_right_logical(cnt + (FCH - 1), 6)
            lax.fori_loop(0, nch, gs_body, 0)
            return carry

        lax.fori_loop(0, NSEG, seg_body, 0)
        plsc.subcore_barrier()
        pltpu.sync_copy(acc_sp.at[pl.ds(RPT * s, RPT)],
                        out_hbm.at[c, r, pl.ds(RPT * s, RPT)])
        plsc.subcore_barrier()


@functools.partial(
    pl.kernel, mesh=_mesh,
    compiler_params=pltpu.CompilerParams(needs_layout_passes=False),
    out_type=[
        jax.ShapeDtypeStruct((NC, R, B, HID), jnp.float32),
        jax.ShapeDtypeStruct((R, B, HID), jnp.float32),
        jax.ShapeDtypeStruct((R, B // 128, 128), jnp.float32),
    ],
    scratch_types=[
        pltpu.VMEM((BPW, HID), jnp.float32),
        pltpu.VMEM((BPW,), jnp.int32),
        pltpu.VMEM((128,), jnp.int32),
        pltpu.VMEM((NROW, 128), jnp.float32),
        pltpu.VMEM((128,), jnp.float32),
        pltpu.SemaphoreType.DMA,
    ],
)
def _tail_gather(acc2_hbm, hs2_hbm, deg_hbm, bidx_hbm, gacc_hbm, ghs2_hbm,
                 gdeg_hbm, rows_v, idx_v, brow_v, degbuf_v, degrow_v, sem):
    c = lax.axis_index("c")
    s = lax.axis_index("s")
    w = c * NS + s
    lane = lax.iota(jnp.int32, 16)
    off = pl.multiple_of(w * BPW, 8)
    pltpu.sync_copy(bidx_hbm.at[pl.ds(off, BPW)], idx_v)
    for r in range(R):
        for cc in range(NC):
            pltpu.async_copy(acc2_hbm.at[cc, r].at[idx_v], rows_v,
                             sem).wait()
            pltpu.sync_copy(rows_v, gacc_hbm.at[cc, r, pl.ds(off, BPW)])
        pltpu.async_copy(hs2_hbm.at[r].at[idx_v], rows_v, sem).wait()
        pltpu.sync_copy(rows_v, ghs2_hbm.at[r, pl.ds(off, BPW)])

    @pl.when(w < B // 128)
    def _():
        doff = pl.multiple_of(w * 128, 8)
        pltpu.sync_copy(bidx_hbm.at[pl.ds(doff, 128)], brow_v)
        for r in range(R):
            pltpu.sync_copy(deg_hbm.at[r], degbuf_v)
            for j in range(8):
                v = brow_v[pl.ds(16 * j, 16)]
                dv = plsc.load_gather(
                    degbuf_v, [lax.shift_right_logical(v, 7),
                               lax.bitwise_and(v, 127)])
                degrow_v[pl.ds(16 * j, 16)] = dv
            pltpu.sync_copy(degrow_v, gdeg_hbm.at[r, w])


_BNROWS = 2000


def _dinv_of(deg_ref):
    return lax.rsqrt(deg_ref[0, 0, 0] + 1.0)


def _deg_reduce_body(h_ref, o_ref):
    k = pl.program_id(2)
    part = jnp.sum(h_ref[...], axis=(0, 1))[0]

    @pl.when(k == 0)
    def _():
        o_ref[0] = part

    @pl.when(k != 0)
    def _():
        o_ref[0] = o_ref[0] + part


def _tc_deg_reduce(hist):
    return pl.pallas_call(
        _deg_reduce_body,
        grid=(R, NROW // 16, NPLN),
        in_specs=[
            pl.BlockSpec((NC, NS, 1, 16, 128),
                         lambda r, j, k: (0, 0, r, (NROW // 16) * k + j, 0)),
        ],
        out_specs=pl.BlockSpec((1, 16, 128), lambda r, j, k: (r, j, 0)),
        out_shape=jax.ShapeDtypeStruct((R, NROW, 128), jnp.float32),
    )(hist)


def _mm1_body(feat_ref, w_ref, deg_ref, hs_ref):
    dinv = _dinv_of(deg_ref)
    h = jnp.dot(feat_ref[...], w_ref[0], preferred_element_type=jnp.float32)
    hs_ref[0] = h * dinv[:, None]


def _mm2_body(acc_ref, hs1_ref, deg_ref, w2_ref, b1_ref, g1_ref, bt1_ref,
              hs2_ref):
    dinv = _dinv_of(deg_ref)
    pre = dinv[:, None] * (acc_ref[0, 0] + acc_ref[1, 0] + hs1_ref[0]) + b1_ref[0]
    x1 = jnp.maximum(pre * (BN_SCALE * g1_ref[0]) + bt1_ref[0], 0.0)
    hs2_ref[0] = jnp.dot(x1, w2_ref[0],
                         preferred_element_type=jnp.float32) * dinv[:, None]


def _tail_body(gacc_ref, ghs2_ref, gdeg_ref, b2_ref, g2_ref, bt2_ref,
               out_ref):
    dinv = lax.rsqrt(gdeg_ref[0, 0] + 1.0)
    zsum = gacc_ref[0, 0] + gacc_ref[1, 0] + ghs2_ref[0]
    z = dinv[:, None] * zsum[:, :OUT] + b2_ref[0]
    y = jnp.maximum(z * (BN_SCALE * g2_ref[0]) + bt2_ref[0], 0.0)
    m = jnp.max(y, axis=-1, keepdims=True)
    e = jnp.exp(y - m)
    out_ref[0] = (y - m) - jnp.log(jnp.sum(e, axis=-1, keepdims=True))


def _tc_hs1(features, W1, deg):
    grid = (R, N // _BNROWS)
    return pl.pallas_call(
        _mm1_body,
        grid=grid,
        in_specs=[
            pl.BlockSpec((_BNROWS, F_IN), lambda r, j: (j, 0)),
            pl.BlockSpec((1, F_IN, HID), lambda r, j: (r, 0, 0)),
            pl.BlockSpec((1, 1, 1, _BNROWS), lambda r, j: (r, j, 0, 0)),
        ],
        out_specs=pl.BlockSpec((1, _BNROWS, HID), lambda r, j: (r, j, 0)),
        out_shape=jax.ShapeDtypeStruct((R, N, HID), jnp.float32),
    )(features, W1, deg)


def _tc_hs2(acc1, hs1, deg, W2, b1, g1, beta1):
    grid = (R, N // _BNROWS)
    return pl.pallas_call(
        _mm2_body,
        grid=grid,
        in_specs=[
            pl.BlockSpec((NC, 1, _BNROWS, HID), lambda r, j: (0, r, j, 0)),
            pl.BlockSpec((1, _BNROWS, HID), lambda r, j: (r, j, 0)),
            pl.BlockSpec((1, 1, 1, _BNROWS), lambda r, j: (r, j, 0, 0)),
            pl.BlockSpec((1, HID, HID), lambda r, j: (r, 0, 0)),
            pl.BlockSpec((1, 1, HID), lambda r, j: (r, 0, 0)),
            pl.BlockSpec((1, 1, HID), lambda r, j: (r, 0, 0)),
            pl.BlockSpec((1, 1, HID), lambda r, j: (r, 0, 0)),
        ],
        out_specs=pl.BlockSpec((1, _BNROWS, HID), lambda r, j: (r, j, 0)),
        out_shape=jax.ShapeDtypeStruct((R, N, HID), jnp.float32),
    )(acc1, hs1, deg, W2, b1, g1, beta1)


def _tc_tail(gacc, ghs2, gdeg, b2, g2, beta2):
    return pl.pallas_call(
        _tail_body,
        grid=(R,),
        in_specs=[
            pl.BlockSpec((NC, 1, B, HID), lambda r: (0, r, 0, 0)),
            pl.BlockSpec((1, B, HID), lambda r: (r, 0, 0)),
            pl.BlockSpec((1, 1, B), lambda r: (r, 0, 0)),
            pl.BlockSpec((1, 1, OUT), lambda r: (r, 0, 0)),
            pl.BlockSpec((1, 1, OUT), lambda r: (r, 0, 0)),
            pl.BlockSpec((1, 1, OUT), lambda r: (r, 0, 0)),
        ],
        out_specs=pl.BlockSpec((1, B, OUT), lambda r: (r, 0, 0)),
        out_shape=jax.ShapeDtypeStruct((R, B, OUT), jnp.float32),
    )(gacc, ghs2, gdeg, b2, g2, beta2)


def _pad_edges(multi_r_data):
    src = multi_r_data[:, 0, :].reshape(R, NW, EPW)
    dst = multi_r_data[:, 1, :].reshape(R, NW, EPW)
    padn = EPW_PAD - EPW
    pad_src = jnp.broadcast_to(
        (jnp.arange(padn, dtype=jnp.int32) * 41) % N, (R, NW, padn))
    pad_dst = jnp.broadcast_to(
        N + jnp.arange(padn, dtype=jnp.int32), (R, NW, padn))
    srcp = jnp.concatenate([src, pad_src], axis=2).reshape(R, NW, NCH, CH)
    dstp = jnp.concatenate([dst, pad_dst], axis=2).reshape(R, NW, NCH, CH)
    return srcp, dstp


def kernel(features, multi_r_data, batch_nodes, W1, b1, g1, beta1,
           W2, b2, g2, beta2):
    zeros_np = jnp.zeros((NP, HID), jnp.float32)
    b1r = b1.reshape(R, 1, HID)
    g1r = g1.reshape(R, 1, HID)
    beta1r = beta1.reshape(R, 1, HID)
    W2p = jnp.pad(W2, ((0, 0), (0, 0), (0, HID - OUT)))
    b2r64 = b2.reshape(R, 1, OUT)
    g2r = g2.reshape(R, 1, OUT)
    beta2r = beta2.reshape(R, 1, OUT)

    srcp, dstp = _pad_edges(multi_r_data)
    hist = _deg_kernel(dstp, zeros_np)
    deg_flat = _tc_deg_reduce(hist)
    deg = deg_flat.reshape(R, NP)[:, :N].reshape(R, N // _BNROWS, 1, _BNROWS)
    hs1 = _tc_hs1(features, W1, deg)
    acc1 = _agg_kernel(hs1, srcp, dstp, zeros_np)
    hs2 = _tc_hs2(acc1, hs1, deg, W2p, b1r, g1r, beta1r)
    zeros_i = jnp.zeros((NROW, 128), jnp.int32)
    acc2 = _agg2_kernel(hs2, srcp, dstp, zeros_np, zeros_i, batch_nodes)
    gacc, ghs2, gdeg = _tail_gather(acc2, hs2, deg_flat, batch_nodes)
    gdeg_lin = gdeg.reshape(R, 1, B)
    out = _tc_tail(gacc, ghs2, gdeg_lin, b2r64, g2r, beta2r)
    return jnp.transpose(out, (1, 0, 2)).reshape(B, R * OUT)
